# Initial kernel scaffold; baseline (speedup 1.0000x reference)
import jax
import jax.numpy as jnp
from jax.experimental import pallas as pl


def kernel(inputs, token_table, pos_table):
    raise NotImplementedError("write your pallas kernel here")


# speedup vs baseline: 2.7061x; 2.7061x over previous
import functools

import jax
import jax.numpy as jnp
from jax import lax
from jax.experimental import pallas as pl
from jax.experimental.pallas import tpu as pltpu
from jax.experimental.pallas import tpu_sc as plsc

NC = 2
NS = 16
NW = NC * NS
NBUF = 4


@functools.lru_cache(maxsize=None)
def _build(B, L, V, D):
    assert B % NW == 0, (B, NW)
    spw = B // NW
    assert spw % NBUF == 0, (spw, NBUF)
    ngrp = spw // NBUF
    assert (L * 4) % 8 == 0 and D % 16 == 0

    mesh = plsc.VectorSubcoreMesh(
        core_axis_name="c", subcore_axis_name="s",
        num_cores=NC, num_subcores=NS)

    scratch_types = (
        [pltpu.VMEM((L, D), jnp.float32)]
        + [pltpu.VMEM((L,), jnp.int32) for _ in range(NBUF)]
        + [pltpu.VMEM((L, D), jnp.float32) for _ in range(NBUF)]
        + [pltpu.SemaphoreType.DMA for _ in range(3 * NBUF)]
    )

    def body(idx_hbm, table_hbm, pos_hbm, out_hbm, *scr):
        pos_v = scr[0]
        idx_v = scr[1:1 + NBUF]
        row_v = scr[1 + NBUF:1 + 2 * NBUF]
        s_idx = scr[1 + 2 * NBUF:1 + 3 * NBUF]
        s_g = scr[1 + 3 * NBUF:1 + 4 * NBUF]
        s_o = scr[1 + 4 * NBUF:1 + 5 * NBUF]

        wid = lax.axis_index("s") * NC + lax.axis_index("c")
        seq0 = wid * spw

        pltpu.sync_copy(pos_hbm, pos_v)

        def start_idx(b, g):
            base = (seq0 + g * NBUF + b) * L
            pltpu.async_copy(idx_hbm.at[pl.ds(base, L)], idx_v[b], s_idx[b])

        def wait_idx(b):
            pltpu.make_async_copy(idx_hbm.at[pl.ds(0, L)], idx_v[b],
                                  s_idx[b]).wait()

        def start_gather(b):
            pltpu.async_copy(table_hbm.at[idx_v[b]], row_v[b], s_g[b])

        def wait_gather(b):
            pltpu.make_async_copy(table_hbm.at[idx_v[b]], row_v[b],
                                  s_g[b]).wait()

        def start_out(b, g):
            base = (seq0 + g * NBUF + b) * L
            pltpu.async_copy(row_v[b], out_hbm.at[pl.ds(base, L)], s_o[b])

        def wait_out(b):
            pltpu.make_async_copy(row_v[b], out_hbm.at[pl.ds(0, L)],
                                  s_o[b]).wait()

        def add_pos(b):
            rv = row_v[b]

            def rbody(r, carry):
                for q in range(D // 16):
                    sl = pl.ds(q * 16, 16)
                    rv[r, sl] = rv[r, sl] + pos_v[r, sl]
                return carry

            lax.fori_loop(0, L, rbody, 0, unroll=2)

        for b in range(NBUF):
            start_idx(b, 0)

        def group(g, carry):
            for b in range(NBUF):
                wait_idx(b)

                @pl.when(g > 0)
                def _():
                    wait_out(b)

                start_gather(b)
            for b in range(NBUF):
                wait_gather(b)
                add_pos(b)
                start_out(b, g)

                @pl.when(g < ngrp - 1)
                def _():
                    start_idx(b, g + 1)
            return carry

        lax.fori_loop(0, ngrp, group, 0)

        for b in range(NBUF):
            wait_out(b)

    return pl.kernel(
        body,
        out_type=jax.ShapeDtypeStruct((B * L, D), jnp.float32),
        mesh=mesh,
        scratch_types=scratch_types,
        compiler_params=pltpu.CompilerParams(use_tc_tiling_on_sc=False),
    )


def kernel(inputs, token_table, pos_table):
    B, L = inputs.shape
    V, D = token_table.shape
    idx = inputs.reshape(-1).astype(jnp.int32)
    out = _build(B, L, V, D)(idx, token_table, pos_table)
    return out.reshape(B, L, D)
